# baseline (speedup 1.0000x reference)
import jax
import jax.numpy as jnp
from jax.experimental import pallas as pl
from jax.experimental.pallas import tpu as pltpu

N = 10000
E = 320000
D_IN = 128
H = 64
RATIO_K = N // 2
NUM_PROP = 2


def _gate_kernel(hk_ref, zk_ref, out_ref):
    out_ref[...] = hk_ref[...] * zk_ref[...]


def kernel(x, edge_index, batch, W1, b1, W2, b2, Ws1, bs1, Ws2, bs2):
    src = edge_index[0]
    dst = edge_index[1]
    agg = jax.ops.segment_sum(x[src], dst, num_segments=N)
    h0 = x + agg
    h = jax.nn.elu(h0 @ W1 + b1)
    h = h @ W2 + b2
    h = jax.nn.elu(h)
    s = jax.nn.elu(h @ Ws1 + bs1)
    ones = jnp.ones((E,), dtype=jnp.float32)
    deg = jax.ops.segment_sum(ones, dst, num_segments=N)
    deg = jnp.clip(deg, 1.0)[:, None]
    for _ in range(NUM_PROP):
        nbr = jax.ops.segment_sum(s[src], dst, num_segments=N)
        s = s - nbr / deg
    z = jnp.tanh(s @ Ws2 + bs2)
    aux_loss = jnp.mean(z[src, 0] * z[dst, 0])
    scores = z[:, 0]
    _, kept_nodes = jax.lax.top_k(scores, RATIO_K)
    hk = h[kept_nodes]
    zk = z[kept_nodes]
    x_pool = pl.pallas_call(
        _gate_kernel,
        out_shape=jax.ShapeDtypeStruct((RATIO_K, H), jnp.float32),
    )(hk, zk)
    return (x_pool, kept_nodes, z, aux_loss)


# speedup vs baseline: 1.1629x; 1.1629x over previous
import functools

import jax
import jax.numpy as jnp
from jax.experimental import pallas as pl
from jax.experimental.pallas import tpu as pltpu
from jax.experimental.pallas import tpu_sc as plsc

_N = 10000
_E = 320000
_D_IN = 128
_H = 64
_K = _N // 2
_NUM_PROP = 2


def _sc_row_gather(table, idx, window):
    B = idx.shape[0]
    D = table.shape[1]
    assert B % window == 0 and window % 8 == 0
    mesh = plsc.VectorSubcoreMesh(core_axis_name="core",
                                  subcore_axis_name="subcore")
    idx2 = idx.reshape(1, B)

    @functools.partial(
        pl.kernel,
        out_type=jax.ShapeDtypeStruct((B, D), table.dtype),
        mesh=mesh,
        compiler_params=pltpu.CompilerParams(use_tc_tiling_on_sc=False),
    )
    def k(x_hbm, i_hbm, o_hbm):
        def body(i_vmem, o_vmem):
            pltpu.sync_copy(x_hbm.at[i_vmem.at[0]], o_vmem)

        pltpu.emit_pipeline(
            body,
            grid=(B // window,),
            in_specs=[pl.BlockSpec((1, window), index_map=lambda i: (0, i))],
            out_specs=[pl.BlockSpec((window, D), index_map=lambda i: (i, 0))],
            core_axis_name=("core", "subcore"),
            dimension_semantics=(pltpu.PARALLEL,),
        )(i_hbm, o_hbm)

    return k(table, idx2)


def _gate_kernel(hk_ref, zk_ref, out_ref):
    out_ref[...] = hk_ref[...] * zk_ref[...]


def kernel(x, edge_index, batch, W1, b1, W2, b2, Ws1, bs1, Ws2, bs2):
    src = edge_index[0]
    dst = edge_index[1]

    xs = _sc_row_gather(x, src, window=256)
    agg = jax.ops.segment_sum(xs, dst, num_segments=_N)
    h0 = x + agg
    h = jax.nn.elu(h0 @ W1 + b1)
    h = h @ W2 + b2
    h = jax.nn.elu(h)

    s = jax.nn.elu(h @ Ws1 + bs1)
    ones = jnp.ones((_E,), dtype=jnp.float32)
    deg = jax.ops.segment_sum(ones, dst, num_segments=_N)
    deg = jnp.clip(deg, 1.0)[:, None]
    for _ in range(_NUM_PROP):
        ss = _sc_row_gather(s, src, window=256)
        nbr = jax.ops.segment_sum(ss, dst, num_segments=_N)
        s = s - nbr / deg
    z = jnp.tanh(s @ Ws2 + bs2)

    aux_loss = jnp.mean(z[src, 0] * z[dst, 0])

    scores = z[:, 0]
    _, kept_nodes = jax.lax.top_k(scores, _K)
    hk = h[kept_nodes]
    zk = z[kept_nodes]
    x_pool = pl.pallas_call(
        _gate_kernel,
        out_shape=jax.ShapeDtypeStruct((_K, _H), jnp.float32),
    )(hk, zk)
    return (x_pool, kept_nodes, z, aux_loss)


# speedup vs baseline: 1.2043x; 1.0357x over previous
import functools

import jax
import jax.numpy as jnp
from jax import lax
from jax.experimental import pallas as pl
from jax.experimental.pallas import tpu as pltpu
from jax.experimental.pallas import tpu_sc as plsc

_N = 10000
_E = 320000
_D_IN = 128
_H = 64
_K = _N // 2
_NUM_PROP = 2
_NW = 32
_C = 80
_SLOTS = 128
_FLUSH_AT = 47

_CUTS_128 = (0, 10080, 20160, 30240, 40320, 50400, 60480, 70560, 80640,
             90720, 100800, 110880, 120720, 130560, 140400, 150240, 160000,
             170080, 180160, 190240, 200320, 210400, 220480, 230560, 240640,
             250720, 260800, 270880, 280720, 290560, 300400, 310240, _E)
_CUTS_64 = (0, 10240, 20480, 30720, 40960, 50880, 60800, 70720, 80640,
            90560, 100480, 110400, 120320, 130240, 140160, 150080, 160000,
            170240, 180480, 190720, 200960, 210880, 220800, 230720, 240640,
            250560, 260480, 270400, 280320, 290240, 300160, 310080, _E)


def _sc_segsum(table, src_s, dst_s, cuts):
    D = table.shape[1]
    KV = D // 16
    n_out = _N + 1 + 2 * _NW
    mesh = plsc.VectorSubcoreMesh(core_axis_name="core",
                                  subcore_axis_name="subcore")
    cuts8 = [0] * 272
    for i, c in enumerate(cuts):
        cuts8[8 * i] = c
    cuts_arr = jnp.array(cuts8, dtype=jnp.int32)

    @functools.partial(
        pl.kernel,
        out_type=jax.ShapeDtypeStruct((n_out, D), jnp.float32),
        mesh=mesh,
        scratch_types=[
            pltpu.VMEM((272,), jnp.int32),
            pltpu.VMEM((16,), jnp.int32),
            pltpu.VMEM((_C,), jnp.int32),
            pltpu.VMEM((_C,), jnp.int32),
            pltpu.VMEM((_C, D), jnp.float32),
            pltpu.VMEM((_SLOTS, D), jnp.float32),
            pltpu.VMEM((_SLOTS,), jnp.int32),
            pltpu.VMEM((_SLOTS,), jnp.int32),
            pltpu.SemaphoreType.DMA,
        ],
        compiler_params=pltpu.CompilerParams(use_tc_tiling_on_sc=False,
                                             needs_layout_passes=False),
    )
    def k(x_hbm, src_hbm, dst_hbm, cuts_hbm, out_hbm,
          cuts_v, prev_v, src_v, dst_v, g_v, loc_v, segid_v, fidx_v, sem):
        wid = lax.axis_index("subcore") * 2 + lax.axis_index("core")
        pltpu.sync_copy(cuts_hbm, cuts_v)
        cw = cuts_v[pl.ds(pl.multiple_of(wid * 8, 8), 16)]
        c_lo = cw[0]
        c_hi = cw[8]
        nchunks = (c_hi - c_lo) // _C
        head_row = _N + 1 + wid
        dump_row = _N + 1 + _NW + wid
        lane = lax.iota(jnp.int32, 16)

        @pl.when(wid > 0)
        def _():
            pltpu.sync_copy(dst_hbm.at[pl.ds(pl.multiple_of(c_lo - 8, 8), 16)],
                            prev_v)

        pv = prev_v[...]
        head_partial = jnp.logical_and(wid > 0, pv[7] == pv[8])

        lane0 = lane == 0

        def flush(cnt, head_done, inclusive):
            lim = cnt + jnp.int32(1) if inclusive else cnt
            for g in range(_SLOTS // 16):
                kvec = lane + jnp.int32(16 * g)
                ids = segid_v[pl.ds(16 * g, 16)]
                valid = kvec < lim
                dest = jnp.where(valid, ids, dump_row)
                if g == 0:
                    divert = jnp.logical_and(
                        jnp.logical_and(head_partial, head_done == 0),
                        lane0)
                    dest = jnp.where(divert, head_row, dest)
                fidx_v[pl.ds(16 * g, 16)] = dest
            pltpu.sync_copy(loc_v, out_hbm.at[fidx_v])

        def store_slot_id(cnt, d):
            plsc.store_scatter(segid_v, [jnp.full((16,), 0, jnp.int32) + cnt],
                               jnp.full((16,), 0, jnp.int32) + d, mask=lane0)

        def group_body(g, st):
            cnt, cur = st[0], st[1]
            accs = list(st[2:])
            off = pl.multiple_of(g * 16, 8)
            dv = dst_v[pl.ds(off, 16)]
            for l in range(16):
                e = g * 16 + l
                d = dv[l]
                chg = d != cur
                cnt = cnt + chg.astype(jnp.int32)
                store_slot_id(cnt, d)
                for kk in range(KV):
                    v = g_v[e, pl.ds(16 * kk, 16)]
                    a = jnp.where(chg, v, accs[kk] + v)
                    loc_v[cnt, pl.ds(16 * kk, 16)] = a
                    accs[kk] = a
                cur = d
            return (cnt, cur) + tuple(accs)

        def chunk_body(j, st):
            cnt, cur, head_done = st[0], st[1], st[2]
            accs = st[3:]
            base = pl.multiple_of(c_lo + j * _C, 8)
            pltpu.sync_copy(src_hbm.at[pl.ds(base, _C)], src_v)
            pltpu.sync_copy(dst_hbm.at[pl.ds(base, _C)], dst_v)
            pltpu.async_copy(x_hbm.at[src_v], g_v, sem).wait()
            st2 = lax.fori_loop(0, _C // 16, group_body, (cnt, cur) + accs)
            cnt, cur = st2[0], st2[1]
            accs = st2[2:]

            def do_flush(args):
                cnt_f, hd = args
                flush(cnt_f, hd, inclusive=False)
                store_slot_id(jnp.int32(0), cur)
                for kk in range(KV):
                    loc_v[0, pl.ds(16 * kk, 16)] = accs[kk]
                return (jnp.int32(0), jnp.int32(1))

            cnt, head_done = lax.cond(cnt >= _FLUSH_AT, do_flush,
                                      lambda a: a, (cnt, head_done))
            return (cnt, cur, head_done) + accs

        zero = jnp.zeros((16,), jnp.float32)
        init = (jnp.int32(-1), jnp.int32(-1), jnp.int32(0)) + (zero,) * KV
        st = lax.fori_loop(0, nchunks, chunk_body, init)
        flush(st[0], st[2], inclusive=True)

    return k(table, src_s, dst_s, cuts_arr)


def _segment_sum_exact(table, src_s, dst_s, cuts, deg_pos):
    D = table.shape[1]
    out = _sc_segsum(table, src_s, dst_s, cuts)
    agg = out[:_N]
    heads = out[_N + 1:_N + 1 + _NW]
    cuts_i = jnp.array(cuts[1:_NW], dtype=jnp.int32)
    head_seg = jnp.take(dst_s, cuts_i)
    prev_seg = jnp.take(dst_s, cuts_i - 1)
    partial = prev_seg == head_seg
    hseg = jnp.where(partial, head_seg, jnp.int32(_N))
    agg_p = jnp.concatenate([agg, jnp.zeros((1, D), jnp.float32)], axis=0)
    for w in range(1, _NW):
        agg_p = agg_p.at[hseg[w - 1]].add(heads[w])
    agg = agg_p[:_N]
    return jnp.where(deg_pos[:, None], agg, 0.0)


def _gate_kernel(hk_ref, zk_ref, out_ref):
    out_ref[...] = hk_ref[...] * zk_ref[...]


def kernel(x, edge_index, batch, W1, b1, W2, b2, Ws1, bs1, Ws2, bs2):
    src = edge_index[0]
    dst = edge_index[1]

    iota = lax.iota(jnp.int32, _E)
    dst_s, perm = lax.sort((dst, iota), num_keys=1, is_stable=True)
    src_s = jnp.take(src, perm)

    ones = jnp.ones((_E,), dtype=jnp.float32)
    deg_raw = jax.ops.segment_sum(ones, dst, num_segments=_N)
    deg_pos = deg_raw > 0.0

    agg = _segment_sum_exact(x, src_s, dst_s, _CUTS_128, deg_pos)
    h0 = x + agg
    h = jax.nn.elu(h0 @ W1 + b1)
    h = h @ W2 + b2
    h = jax.nn.elu(h)

    s = jax.nn.elu(h @ Ws1 + bs1)
    deg = jnp.clip(deg_raw, 1.0)[:, None]
    for _ in range(_NUM_PROP):
        nbr = _segment_sum_exact(s, src_s, dst_s, _CUTS_64, deg_pos)
        s = s - nbr / deg
    z = jnp.tanh(s @ Ws2 + bs2)

    aux_loss = jnp.mean(z[src, 0] * z[dst, 0])

    scores = z[:, 0]
    _, kept_nodes = jax.lax.top_k(scores, _K)
    hk = h[kept_nodes]
    zk = z[kept_nodes]
    x_pool = pl.pallas_call(
        _gate_kernel,
        out_shape=jax.ShapeDtypeStruct((_K, _H), jnp.float32),
    )(hk, zk)
    return (x_pool, kept_nodes, z, aux_loss)


# speedup vs baseline: 3.7673x; 3.1281x over previous
import functools

import jax
import jax.numpy as jnp
from jax import lax
from jax.experimental import pallas as pl
from jax.experimental.pallas import tpu as pltpu
from jax.experimental.pallas import tpu_sc as plsc

_N = 10000
_E = 320000
_D_IN = 128
_H = 64
_K = _N // 2
_NUM_PROP = 2
_NW = 32
_C = 80
_SLOTS = 128
_FLUSH_AT = 47

_CUTS_128 = (0, 10080, 20160, 30240, 40320, 50400, 60480, 70560, 80640,
             90720, 100800, 110880, 120720, 130560, 140400, 150240, 160000,
             170080, 180160, 190240, 200320, 210400, 220480, 230560, 240640,
             250720, 260800, 270880, 280720, 290560, 300400, 310240, _E)
_CUTS_64 = (0, 10240, 20480, 30720, 40960, 50880, 60800, 70720, 80640,
            90560, 100480, 110400, 120320, 130240, 140160, 150080, 160000,
            170240, 180480, 190720, 200960, 210880, 220800, 230720, 240640,
            250560, 260480, 270400, 280320, 290240, 300160, 310080, _E)


def _sc_segsum(table, src_s, dst_s, cuts):
    D = table.shape[1]
    KV = D // 16
    n_out = _N + 1 + 2 * _NW
    mesh = plsc.VectorSubcoreMesh(core_axis_name="core",
                                  subcore_axis_name="subcore")
    cuts8 = [0] * 272
    for i, c in enumerate(cuts):
        cuts8[8 * i] = c
    cuts_arr = jnp.array(cuts8, dtype=jnp.int32)

    @functools.partial(
        pl.kernel,
        out_type=jax.ShapeDtypeStruct((n_out, D), jnp.float32),
        mesh=mesh,
        scratch_types=[
            pltpu.VMEM((272,), jnp.int32),
            pltpu.VMEM((16,), jnp.int32),
            pltpu.VMEM((_C,), jnp.int32),
            pltpu.VMEM((_C,), jnp.int32),
            pltpu.VMEM((_C, D), jnp.float32),
            pltpu.VMEM((_SLOTS, D), jnp.float32),
            pltpu.VMEM((_SLOTS,), jnp.int32),
            pltpu.VMEM((_SLOTS,), jnp.int32),
            pltpu.SemaphoreType.DMA,
        ],
        compiler_params=pltpu.CompilerParams(use_tc_tiling_on_sc=False,
                                             needs_layout_passes=False),
    )
    def k(x_hbm, src_hbm, dst_hbm, cuts_hbm, out_hbm,
          cuts_v, prev_v, src_v, dst_v, g_v, loc_v, segid_v, fidx_v, sem):
        wid = lax.axis_index("subcore") * 2 + lax.axis_index("core")
        pltpu.sync_copy(cuts_hbm, cuts_v)
        cw = cuts_v[pl.ds(pl.multiple_of(wid * 8, 8), 16)]
        c_lo = cw[0]
        c_hi = cw[8]
        nchunks = (c_hi - c_lo) // _C
        head_row = _N + 1 + wid
        dump_row = _N + 1 + _NW + wid
        lane = lax.iota(jnp.int32, 16)

        @pl.when(wid > 0)
        def _():
            pltpu.sync_copy(dst_hbm.at[pl.ds(pl.multiple_of(c_lo - 8, 8), 16)],
                            prev_v)

        pv = prev_v[...]
        head_partial = jnp.logical_and(wid > 0, pv[7] == pv[8])

        lane0 = lane == 0

        def flush(cnt, head_done, inclusive):
            lim = cnt + jnp.int32(1) if inclusive else cnt
            for g in range(_SLOTS // 16):
                kvec = lane + jnp.int32(16 * g)
                ids = segid_v[pl.ds(16 * g, 16)]
                valid = kvec < lim
                dest = jnp.where(valid, ids, dump_row)
                if g == 0:
                    divert = jnp.logical_and(
                        jnp.logical_and(head_partial, head_done == 0),
                        lane0)
                    dest = jnp.where(divert, head_row, dest)
                fidx_v[pl.ds(16 * g, 16)] = dest
            pltpu.sync_copy(loc_v, out_hbm.at[fidx_v])

        def store_slot_id(cnt, d):
            plsc.store_scatter(segid_v, [jnp.full((16,), 0, jnp.int32) + cnt],
                               jnp.full((16,), 0, jnp.int32) + d, mask=lane0)

        def group_body(g, st):
            cnt, cur = st[0], st[1]
            accs = list(st[2:])
            off = pl.multiple_of(g * 16, 8)
            dv = dst_v[pl.ds(off, 16)]
            for l in range(16):
                e = g * 16 + l
                d = dv[l]
                chg = d != cur
                cnt = cnt + chg.astype(jnp.int32)
                store_slot_id(cnt, d)
                for kk in range(KV):
                    v = g_v[e, pl.ds(16 * kk, 16)]
                    a = jnp.where(chg, v, accs[kk] + v)
                    loc_v[cnt, pl.ds(16 * kk, 16)] = a
                    accs[kk] = a
                cur = d
            return (cnt, cur) + tuple(accs)

        def chunk_body(j, st):
            cnt, cur, head_done = st[0], st[1], st[2]
            accs = st[3:]
            base = pl.multiple_of(c_lo + j * _C, 8)
            pltpu.sync_copy(src_hbm.at[pl.ds(base, _C)], src_v)
            pltpu.sync_copy(dst_hbm.at[pl.ds(base, _C)], dst_v)
            pltpu.async_copy(x_hbm.at[src_v], g_v, sem).wait()
            st2 = lax.fori_loop(0, _C // 16, group_body, (cnt, cur) + accs)
            cnt, cur = st2[0], st2[1]
            accs = st2[2:]

            def do_flush(args):
                cnt_f, hd = args
                flush(cnt_f, hd, inclusive=False)
                store_slot_id(jnp.int32(0), cur)
                for kk in range(KV):
                    loc_v[0, pl.ds(16 * kk, 16)] = accs[kk]
                return (jnp.int32(0), jnp.int32(1))

            cnt, head_done = lax.cond(cnt >= _FLUSH_AT, do_flush,
                                      lambda a: a, (cnt, head_done))
            return (cnt, cur, head_done) + accs

        zero = jnp.zeros((16,), jnp.float32)
        init = (jnp.int32(-1), jnp.int32(-1), jnp.int32(0)) + (zero,) * KV
        st = lax.fori_loop(0, nchunks, chunk_body, init)
        flush(st[0], st[2], inclusive=True)

    return k(table, src_s, dst_s, cuts_arr)


def _segment_sum_exact(table, src_s, dst_s, cuts, deg_pos):
    D = table.shape[1]
    out = _sc_segsum(table, src_s, dst_s, cuts)
    agg = out[:_N]
    heads = out[_N + 1:_N + 1 + _NW]
    cuts_i = jnp.array(cuts[1:_NW], dtype=jnp.int32)
    head_seg = jnp.take(dst_s, cuts_i)
    prev_seg = jnp.take(dst_s, cuts_i - 1)
    partial = prev_seg == head_seg
    hseg = jnp.where(partial, head_seg, jnp.int32(_N))
    agg_p = jnp.concatenate([agg, jnp.zeros((1, D), jnp.float32)], axis=0)
    for w in range(1, _NW):
        agg_p = agg_p.at[hseg[w - 1]].add(heads[w])
    agg = agg_p[:_N]
    return jnp.where(deg_pos[:, None], agg, 0.0)


def _sc_aux_partials(z_flat, src, dst):
    epw = _E // _NW
    mesh = plsc.VectorSubcoreMesh(core_axis_name="core",
                                  subcore_axis_name="subcore")

    @functools.partial(
        pl.kernel,
        out_type=jax.ShapeDtypeStruct((_NW, 16), jnp.float32),
        mesh=mesh,
        scratch_types=[
            pltpu.VMEM((_N,), jnp.float32),
            pltpu.VMEM((epw,), jnp.int32),
            pltpu.VMEM((epw,), jnp.int32),
            pltpu.VMEM((16,), jnp.float32),
        ],
        compiler_params=pltpu.CompilerParams(use_tc_tiling_on_sc=False,
                                             needs_layout_passes=False),
    )
    def k(z_hbm, src_hbm, dst_hbm, out_hbm, z_v, src_v, dst_v, acc_v):
        wid = lax.axis_index("subcore") * 2 + lax.axis_index("core")
        base = pl.multiple_of(wid * epw, 8)
        pltpu.sync_copy(z_hbm, z_v)
        pltpu.sync_copy(src_hbm.at[pl.ds(base, epw)], src_v)
        pltpu.sync_copy(dst_hbm.at[pl.ds(base, epw)], dst_v)

        def body(i, acc):
            off = pl.multiple_of(i * 16, 8)
            si = src_v[pl.ds(off, 16)]
            di = dst_v[pl.ds(off, 16)]
            zs = plsc.load_gather(z_v, [si])
            zd = plsc.load_gather(z_v, [di])
            return acc + zs * zd

        acc = lax.fori_loop(0, epw // 16, body,
                            jnp.zeros((16,), jnp.float32))
        acc_v[...] = acc
        pltpu.sync_copy(acc_v, out_hbm.at[wid])

    return k(z_flat, src, dst)


def _gate_kernel(hk_ref, zk_ref, out_ref):
    out_ref[...] = hk_ref[...] * zk_ref[...]


def kernel(x, edge_index, batch, W1, b1, W2, b2, Ws1, bs1, Ws2, bs2):
    src = edge_index[0]
    dst = edge_index[1]

    iota = lax.iota(jnp.int32, _E)
    dst_s, perm = lax.sort((dst, iota), num_keys=1, is_stable=True)
    src_s = jnp.take(src, perm)

    ones = jnp.ones((_E,), dtype=jnp.float32)
    deg_raw = jax.ops.segment_sum(ones, dst, num_segments=_N)
    deg_pos = deg_raw > 0.0

    agg = _segment_sum_exact(x, src_s, dst_s, _CUTS_128, deg_pos)
    h0 = x + agg
    h = jax.nn.elu(h0 @ W1 + b1)
    h = h @ W2 + b2
    h = jax.nn.elu(h)

    s = jax.nn.elu(h @ Ws1 + bs1)
    deg = jnp.clip(deg_raw, 1.0)[:, None]
    for _ in range(_NUM_PROP):
        nbr = _segment_sum_exact(s, src_s, dst_s, _CUTS_64, deg_pos)
        s = s - nbr / deg
    z = jnp.tanh(s @ Ws2 + bs2)

    aux_parts = _sc_aux_partials(z[:, 0], src, dst)
    aux_loss = jnp.sum(aux_parts) / jnp.float32(_E)

    scores = z[:, 0]
    _, kept_nodes = jax.lax.top_k(scores, _K)
    hk = h[kept_nodes]
    zk = z[kept_nodes]
    x_pool = pl.pallas_call(
        _gate_kernel,
        out_shape=jax.ShapeDtypeStruct((_K, _H), jnp.float32),
    )(hk, zk)
    return (x_pool, kept_nodes, z, aux_loss)


# speedup vs baseline: 4.4706x; 1.1867x over previous
import functools

import jax
import jax.numpy as jnp
from jax import lax
from jax.experimental import pallas as pl
from jax.experimental.pallas import tpu as pltpu
from jax.experimental.pallas import tpu_sc as plsc

_N = 10000
_E = 320000
_D_IN = 128
_H = 64
_K = _N // 2
_NUM_PROP = 2
_NW = 32
_C = 80
_CBIG = 400
_SLOTS = 128
_FLUSH_AT = 47

_CUTS_128 = (0, 10080, 20160, 30240, 40320, 50400, 60480, 70560, 80640,
             90720, 100800, 110880, 120720, 130560, 140400, 150240, 160000,
             170080, 180160, 190240, 200320, 210400, 220480, 230560, 240640,
             250720, 260800, 270880, 280720, 290560, 300400, 310240, _E)
_CUTS_64 = (0, 10240, 20480, 30720, 40960, 50880, 60800, 70720, 80640,
            90560, 100480, 110400, 120320, 130240, 140160, 150080, 160000,
            170240, 180480, 190720, 200960, 210880, 220800, 230720, 240640,
            250560, 260480, 270400, 280320, 290240, 300160, 310080, _E)


def _sc_segsum(table, src_s, dst_s, cuts):
    D = table.shape[1]
    KV = D // 16
    n_out = _N + 1 + 2 * _NW
    mesh = plsc.VectorSubcoreMesh(core_axis_name="core",
                                  subcore_axis_name="subcore")
    cuts8 = [0] * 272
    for i, c in enumerate(cuts):
        cuts8[8 * i] = c
    cuts_arr = jnp.array(cuts8, dtype=jnp.int32)

    @functools.partial(
        pl.kernel,
        out_type=jax.ShapeDtypeStruct((n_out, D), jnp.float32),
        mesh=mesh,
        scratch_types=[
            pltpu.VMEM((272,), jnp.int32),
            pltpu.VMEM((16,), jnp.int32),
            pltpu.VMEM((_CBIG,), jnp.int32),
            pltpu.VMEM((_CBIG,), jnp.int32),
            pltpu.VMEM((_CBIG, D), jnp.float32),
            pltpu.VMEM((_SLOTS, D), jnp.float32),
            pltpu.VMEM((_SLOTS,), jnp.int32),
            pltpu.VMEM((_SLOTS,), jnp.int32),
            pltpu.SemaphoreType.DMA,
        ],
        compiler_params=pltpu.CompilerParams(use_tc_tiling_on_sc=False,
                                             needs_layout_passes=False),
    )
    def k(x_hbm, src_hbm, dst_hbm, cuts_hbm, out_hbm,
          cuts_v, prev_v, src_v, dst_v, g_v, loc_v, segid_v, fidx_v, sem):
        wid = lax.axis_index("subcore") * 2 + lax.axis_index("core")
        pltpu.sync_copy(cuts_hbm, cuts_v)
        cw = cuts_v[pl.ds(pl.multiple_of(wid * 8, 8), 16)]
        c_lo = cw[0]
        c_hi = cw[8]
        nchunks = (c_hi - c_lo) // _C
        head_row = _N + 1 + wid
        dump_row = _N + 1 + _NW + wid
        lane = lax.iota(jnp.int32, 16)

        @pl.when(wid > 0)
        def _():
            pltpu.sync_copy(dst_hbm.at[pl.ds(pl.multiple_of(c_lo - 8, 8), 16)],
                            prev_v)

        pv = prev_v[...]
        head_partial = jnp.logical_and(wid > 0, pv[7] == pv[8])

        lane0 = lane == 0

        def flush(cnt, head_done, inclusive):
            lim = cnt + jnp.int32(1) if inclusive else cnt
            for g in range(_SLOTS // 16):
                kvec = lane + jnp.int32(16 * g)
                ids = segid_v[pl.ds(16 * g, 16)]
                valid = kvec < lim
                dest = jnp.where(valid, ids, dump_row)
                if g == 0:
                    divert = jnp.logical_and(
                        jnp.logical_and(head_partial, head_done == 0),
                        lane0)
                    dest = jnp.where(divert, head_row, dest)
                fidx_v[pl.ds(16 * g, 16)] = dest
            pltpu.sync_copy(loc_v, out_hbm.at[fidx_v])

        def store_slot_id(cnt, d):
            plsc.store_scatter(segid_v, [jnp.full((16,), 0, jnp.int32) + cnt],
                               jnp.full((16,), 0, jnp.int32) + d, mask=lane0)

        def group_body(g, st):
            cnt, cur, head_done = st[0], st[1], st[2]
            accs = list(st[3:])
            off = pl.multiple_of(g * 16, 8)
            dv = dst_v[pl.ds(off, 16)]
            for l in range(16):
                e = g * 16 + l
                d = dv[l]
                chg = d != cur
                cnt = cnt + chg.astype(jnp.int32)
                store_slot_id(cnt, d)
                for kk in range(KV):
                    v = g_v[e, pl.ds(16 * kk, 16)]
                    a = jnp.where(chg, v, accs[kk] + v)
                    loc_v[cnt, pl.ds(16 * kk, 16)] = a
                    accs[kk] = a
                cur = d

            def do_flush(args):
                cnt_f, hd = args
                flush(cnt_f, hd, inclusive=False)
                store_slot_id(jnp.int32(0), cur)
                for kk in range(KV):
                    loc_v[0, pl.ds(16 * kk, 16)] = accs[kk]
                return (jnp.int32(0), jnp.int32(1))

            cnt, head_done = lax.cond(cnt >= _SLOTS - 17, do_flush,
                                      lambda a: a, (cnt, head_done))
            return (cnt, cur, head_done) + tuple(accs)

        def make_chunk_body(csize, base_off):
            def chunk_body(j, st):
                base = pl.multiple_of(c_lo + base_off + j * csize, 8)
                pltpu.sync_copy(src_hbm.at[pl.ds(base, csize)],
                                src_v.at[pl.ds(0, csize)])
                pltpu.sync_copy(dst_hbm.at[pl.ds(base, csize)],
                                dst_v.at[pl.ds(0, csize)])
                pltpu.async_copy(x_hbm.at[src_v.at[pl.ds(0, csize)]],
                                 g_v.at[pl.ds(0, csize)], sem).wait()
                return lax.fori_loop(0, csize // 16, group_body, st)
            return chunk_body

        nfull = (c_hi - c_lo) // _CBIG
        rem = (c_hi - c_lo) - nfull * _CBIG
        ntail = rem // _C
        zero = jnp.zeros((16,), jnp.float32)
        init = (jnp.int32(-1), jnp.int32(-1), jnp.int32(0)) + (zero,) * KV
        st = lax.fori_loop(0, nfull, make_chunk_body(_CBIG, 0), init)
        st = lax.fori_loop(0, ntail,
                           make_chunk_body(_C, nfull * _CBIG), st)
        flush(st[0], st[2], inclusive=True)

    return k(table, src_s, dst_s, cuts_arr)


def _segment_sum_exact(table, src_s, dst_s, cuts, deg_pos):
    D = table.shape[1]
    out = _sc_segsum(table, src_s, dst_s, cuts)
    agg = out[:_N]
    heads = out[_N + 1:_N + 1 + _NW]
    cuts_i = jnp.array(cuts[1:_NW], dtype=jnp.int32)
    head_seg = jnp.take(dst_s, cuts_i)
    prev_seg = jnp.take(dst_s, cuts_i - 1)
    partial = prev_seg == head_seg
    hseg = jnp.where(partial, head_seg, jnp.int32(_N))
    agg_p = jnp.concatenate([agg, jnp.zeros((1, D), jnp.float32)], axis=0)
    for w in range(1, _NW):
        agg_p = agg_p.at[hseg[w - 1]].add(heads[w])
    agg = agg_p[:_N]
    return jnp.where(deg_pos[:, None], agg, 0.0)


def _sc_aux_partials(z_flat, src, dst):
    epw = _E // _NW
    mesh = plsc.VectorSubcoreMesh(core_axis_name="core",
                                  subcore_axis_name="subcore")

    @functools.partial(
        pl.kernel,
        out_type=jax.ShapeDtypeStruct((_NW, 16), jnp.float32),
        mesh=mesh,
        scratch_types=[
            pltpu.VMEM((_N,), jnp.float32),
            pltpu.VMEM((epw,), jnp.int32),
            pltpu.VMEM((epw,), jnp.int32),
            pltpu.VMEM((16,), jnp.float32),
        ],
        compiler_params=pltpu.CompilerParams(use_tc_tiling_on_sc=False,
                                             needs_layout_passes=False),
    )
    def k(z_hbm, src_hbm, dst_hbm, out_hbm, z_v, src_v, dst_v, acc_v):
        wid = lax.axis_index("subcore") * 2 + lax.axis_index("core")
        base = pl.multiple_of(wid * epw, 8)
        pltpu.sync_copy(z_hbm, z_v)
        pltpu.sync_copy(src_hbm.at[pl.ds(base, epw)], src_v)
        pltpu.sync_copy(dst_hbm.at[pl.ds(base, epw)], dst_v)

        def body(i, acc):
            off = pl.multiple_of(i * 16, 8)
            si = src_v[pl.ds(off, 16)]
            di = dst_v[pl.ds(off, 16)]
            zs = plsc.load_gather(z_v, [si])
            zd = plsc.load_gather(z_v, [di])
            return acc + zs * zd

        acc = lax.fori_loop(0, epw // 16, body,
                            jnp.zeros((16,), jnp.float32))
        acc_v[...] = acc
        pltpu.sync_copy(acc_v, out_hbm.at[wid])

    return k(z_flat, src, dst)


def _gate_kernel(hk_ref, zk_ref, out_ref):
    out_ref[...] = hk_ref[...] * zk_ref[...]


def kernel(x, edge_index, batch, W1, b1, W2, b2, Ws1, bs1, Ws2, bs2):
    src = edge_index[0]
    dst = edge_index[1]

    iota = lax.iota(jnp.int32, _E)
    dst_s, perm = lax.sort((dst, iota), num_keys=1, is_stable=True)
    src_s = jnp.take(src, perm)

    ones = jnp.ones((_E,), dtype=jnp.float32)
    deg_raw = jax.ops.segment_sum(ones, dst, num_segments=_N)
    deg_pos = deg_raw > 0.0

    agg = _segment_sum_exact(x, src_s, dst_s, _CUTS_128, deg_pos)
    h0 = x + agg
    h = jax.nn.elu(h0 @ W1 + b1)
    h = h @ W2 + b2
    h = jax.nn.elu(h)

    s = jax.nn.elu(h @ Ws1 + bs1)
    deg = jnp.clip(deg_raw, 1.0)[:, None]
    for _ in range(_NUM_PROP):
        nbr = _segment_sum_exact(s, src_s, dst_s, _CUTS_64, deg_pos)
        s = s - nbr / deg
    z = jnp.tanh(s @ Ws2 + bs2)

    aux_parts = _sc_aux_partials(z[:, 0], src, dst)
    aux_loss = jnp.sum(aux_parts) / jnp.float32(_E)

    scores = z[:, 0]
    _, kept_nodes = jax.lax.top_k(scores, _K)
    hk = h[kept_nodes]
    zk = z[kept_nodes]
    x_pool = pl.pallas_call(
        _gate_kernel,
        out_shape=jax.ShapeDtypeStruct((_K, _H), jnp.float32),
    )(hk, zk)
    return (x_pool, kept_nodes, z, aux_loss)


# speedup vs baseline: 4.5977x; 1.0284x over previous
import functools

import jax
import jax.numpy as jnp
from jax import lax
from jax.experimental import pallas as pl
from jax.experimental.pallas import tpu as pltpu
from jax.experimental.pallas import tpu_sc as plsc

_N = 10000
_E = 320000
_D_IN = 128
_H = 64
_K = _N // 2
_NUM_PROP = 2
_NW = 32
_C = 80
_CBIG = 560
_RMAX = 10240
_SLOTS = 128
_FLUSH_AT = 47

_CUTS_128 = (0, 10080, 20160, 30240, 40320, 50400, 60480, 70560, 80640,
             90720, 100800, 110880, 120720, 130560, 140400, 150240, 160000,
             170080, 180160, 190240, 200320, 210400, 220480, 230560, 240640,
             250720, 260800, 270880, 280720, 290560, 300400, 310240, _E)
_CUTS_64 = (0, 10240, 20480, 30720, 40960, 50880, 60800, 70720, 80640,
            90560, 100480, 110400, 120320, 130240, 140160, 150080, 160000,
            170240, 180480, 190720, 200960, 210880, 220800, 230720, 240640,
            250560, 260480, 270400, 280320, 290240, 300160, 310080, _E)


def _sc_segsum(table, src_s, dst_s, cuts):
    D = table.shape[1]
    KV = D // 16
    n_out = _N + 1 + 2 * _NW
    mesh = plsc.VectorSubcoreMesh(core_axis_name="core",
                                  subcore_axis_name="subcore")
    cuts8 = [0] * 272
    for i, c in enumerate(cuts):
        cuts8[8 * i] = c
    cuts_arr = jnp.array(cuts8, dtype=jnp.int32)

    @functools.partial(
        pl.kernel,
        out_type=jax.ShapeDtypeStruct((n_out, D), jnp.float32),
        mesh=mesh,
        scratch_types=[
            pltpu.VMEM((272,), jnp.int32),
            pltpu.VMEM((16,), jnp.int32),
            pltpu.VMEM((_RMAX,), jnp.int32),
            pltpu.VMEM((_RMAX,), jnp.int32),
            pltpu.VMEM((_CBIG, D), jnp.float32),
            pltpu.VMEM((_SLOTS, D), jnp.float32),
            pltpu.VMEM((_SLOTS,), jnp.int32),
            pltpu.VMEM((_SLOTS,), jnp.int32),
            pltpu.SemaphoreType.DMA,
        ],
        compiler_params=pltpu.CompilerParams(use_tc_tiling_on_sc=False,
                                             needs_layout_passes=False),
    )
    def k(x_hbm, src_hbm, dst_hbm, cuts_hbm, out_hbm,
          cuts_v, prev_v, src_v, dst_v, g_v, loc_v, segid_v, fidx_v, sem):
        wid = lax.axis_index("subcore") * 2 + lax.axis_index("core")
        pltpu.sync_copy(cuts_hbm, cuts_v)
        cw = cuts_v[pl.ds(pl.multiple_of(wid * 8, 8), 16)]
        c_lo = cw[0]
        c_hi = cw[8]
        nchunks = (c_hi - c_lo) // _C
        head_row = _N + 1 + wid
        dump_row = _N + 1 + _NW + wid
        lane = lax.iota(jnp.int32, 16)

        @pl.when(wid > 0)
        def _():
            pltpu.sync_copy(dst_hbm.at[pl.ds(pl.multiple_of(c_lo - 8, 8), 16)],
                            prev_v)

        pv = prev_v[...]
        head_partial = jnp.logical_and(wid > 0, pv[7] == pv[8])

        lane0 = lane == 0

        def flush(cnt, head_done, inclusive):
            lim = cnt + jnp.int32(1) if inclusive else cnt
            for g in range(_SLOTS // 16):
                kvec = lane + jnp.int32(16 * g)
                ids = segid_v[pl.ds(16 * g, 16)]
                valid = kvec < lim
                dest = jnp.where(valid, ids, dump_row)
                if g == 0:
                    divert = jnp.logical_and(
                        jnp.logical_and(head_partial, head_done == 0),
                        lane0)
                    dest = jnp.where(divert, head_row, dest)
                fidx_v[pl.ds(16 * g, 16)] = dest
            pltpu.sync_copy(loc_v, out_hbm.at[fidx_v])

        def store_slot_id(cnt, d):
            plsc.store_scatter(segid_v, [jnp.full((16,), 0, jnp.int32) + cnt],
                               jnp.full((16,), 0, jnp.int32) + d, mask=lane0)

        pltpu.sync_copy(src_hbm.at[pl.ds(pl.multiple_of(c_lo, 8), _RMAX)],
                        src_v)
        pltpu.sync_copy(dst_hbm.at[pl.ds(pl.multiple_of(c_lo, 8), _RMAX)],
                        dst_v)

        def make_chunk_body(csize, base_off):
            def chunk_body(j, st):
                loff = base_off + j * csize
                pltpu.async_copy(
                    x_hbm.at[src_v.at[pl.ds(pl.multiple_of(loff, 8), csize)]],
                    g_v.at[pl.ds(0, csize)], sem).wait()

                def group_body(g, st2):
                    cnt, cur, head_done = st2[0], st2[1], st2[2]
                    accs = list(st2[3:])
                    off = pl.multiple_of(loff + g * 16, 8)
                    dv = dst_v[pl.ds(off, 16)]
                    for l in range(16):
                        e = g * 16 + l
                        d = dv[l]
                        chg = d != cur
                        cnt = cnt + chg.astype(jnp.int32)
                        store_slot_id(cnt, d)
                        for kk in range(KV):
                            v = g_v[e, pl.ds(16 * kk, 16)]
                            a = jnp.where(chg, v, accs[kk] + v)
                            loc_v[cnt, pl.ds(16 * kk, 16)] = a
                            accs[kk] = a
                        cur = d

                    def do_flush(args):
                        cnt_f, hd = args
                        flush(cnt_f, hd, inclusive=False)
                        store_slot_id(jnp.int32(0), cur)
                        for kk in range(KV):
                            loc_v[0, pl.ds(16 * kk, 16)] = accs[kk]
                        return (jnp.int32(0), jnp.int32(1))

                    cnt, head_done = lax.cond(cnt >= _SLOTS - 17, do_flush,
                                              lambda a: a, (cnt, head_done))
                    return (cnt, cur, head_done) + tuple(accs)

                return lax.fori_loop(0, csize // 16, group_body, st)
            return chunk_body

        nfull = (c_hi - c_lo) // _CBIG
        rem = (c_hi - c_lo) - nfull * _CBIG
        ntail = rem // _C
        zero = jnp.zeros((16,), jnp.float32)
        init = (jnp.int32(-1), jnp.int32(-1), jnp.int32(0)) + (zero,) * KV
        st = lax.fori_loop(0, nfull, make_chunk_body(_CBIG, 0), init)
        st = lax.fori_loop(0, ntail,
                           make_chunk_body(_C, nfull * _CBIG), st)
        flush(st[0], st[2], inclusive=True)

    return k(table, src_s, dst_s, cuts_arr)


def _segment_sum_exact(table, src_sp, dst_sp, dst_s, cuts, deg_pos):
    D = table.shape[1]
    out = _sc_segsum(table, src_sp, dst_sp, cuts)
    agg = out[:_N]
    heads = out[_N + 1:_N + 1 + _NW]
    cuts_i = jnp.array(cuts[1:_NW], dtype=jnp.int32)
    head_seg = jnp.take(dst_s, cuts_i)
    prev_seg = jnp.take(dst_s, cuts_i - 1)
    partial = prev_seg == head_seg
    hseg = jnp.where(partial, head_seg, jnp.int32(_N))
    agg_p = jnp.concatenate([agg, jnp.zeros((1, D), jnp.float32)], axis=0)
    for w in range(1, _NW):
        agg_p = agg_p.at[hseg[w - 1]].add(heads[w])
    agg = agg_p[:_N]
    return jnp.where(deg_pos[:, None], agg, 0.0)


def _sc_aux_partials(z_flat, src, dst):
    epw = _E // _NW
    mesh = plsc.VectorSubcoreMesh(core_axis_name="core",
                                  subcore_axis_name="subcore")

    @functools.partial(
        pl.kernel,
        out_type=jax.ShapeDtypeStruct((_NW, 16), jnp.float32),
        mesh=mesh,
        scratch_types=[
            pltpu.VMEM((_N,), jnp.float32),
            pltpu.VMEM((epw,), jnp.int32),
            pltpu.VMEM((epw,), jnp.int32),
            pltpu.VMEM((16,), jnp.float32),
        ],
        compiler_params=pltpu.CompilerParams(use_tc_tiling_on_sc=False,
                                             needs_layout_passes=False),
    )
    def k(z_hbm, src_hbm, dst_hbm, out_hbm, z_v, src_v, dst_v, acc_v):
        wid = lax.axis_index("subcore") * 2 + lax.axis_index("core")
        base = pl.multiple_of(wid * epw, 8)
        pltpu.sync_copy(z_hbm, z_v)
        pltpu.sync_copy(src_hbm.at[pl.ds(base, epw)], src_v)
        pltpu.sync_copy(dst_hbm.at[pl.ds(base, epw)], dst_v)

        def body(i, acc):
            off = pl.multiple_of(i * 16, 8)
            si = src_v[pl.ds(off, 16)]
            di = dst_v[pl.ds(off, 16)]
            zs = plsc.load_gather(z_v, [si])
            zd = plsc.load_gather(z_v, [di])
            return acc + zs * zd

        acc = lax.fori_loop(0, epw // 16, body,
                            jnp.zeros((16,), jnp.float32))
        acc_v[...] = acc
        pltpu.sync_copy(acc_v, out_hbm.at[wid])

    return k(z_flat, src, dst)


def _gate_kernel(hk_ref, zk_ref, out_ref):
    out_ref[...] = hk_ref[...] * zk_ref[...]


def kernel(x, edge_index, batch, W1, b1, W2, b2, Ws1, bs1, Ws2, bs2):
    src = edge_index[0]
    dst = edge_index[1]

    iota = lax.iota(jnp.int32, _E)
    dst_s, perm = lax.sort((dst, iota), num_keys=1, is_stable=True)
    src_s = jnp.take(src, perm)
    zpad = jnp.zeros((512,), jnp.int32)
    src_sp = jnp.concatenate([src_s, zpad])
    dst_sp = jnp.concatenate([dst_s, zpad])

    ones = jnp.ones((_E,), dtype=jnp.float32)
    deg_raw = jax.ops.segment_sum(ones, dst, num_segments=_N)
    deg_pos = deg_raw > 0.0

    agg = _segment_sum_exact(x, src_sp, dst_sp, dst_s, _CUTS_128, deg_pos)
    h0 = x + agg
    h = jax.nn.elu(h0 @ W1 + b1)
    h = h @ W2 + b2
    h = jax.nn.elu(h)

    s = jax.nn.elu(h @ Ws1 + bs1)
    deg = jnp.clip(deg_raw, 1.0)[:, None]
    for _ in range(_NUM_PROP):
        nbr = _segment_sum_exact(s, src_sp, dst_sp, dst_s, _CUTS_64, deg_pos)
        s = s - nbr / deg
    z = jnp.tanh(s @ Ws2 + bs2)

    aux_parts = _sc_aux_partials(z[:, 0], src, dst)
    aux_loss = jnp.sum(aux_parts) / jnp.float32(_E)

    scores = z[:, 0]
    _, kept_nodes = jax.lax.top_k(scores, _K)
    hk = h[kept_nodes]
    zk = z[kept_nodes]
    x_pool = pl.pallas_call(
        _gate_kernel,
        out_shape=jax.ShapeDtypeStruct((_K, _H), jnp.float32),
    )(hk, zk)
    return (x_pool, kept_nodes, z, aux_loss)
